# baseline (speedup 1.0000x reference)
import functools
import math

import jax
import jax.numpy as jnp
from jax import lax
from jax.experimental import pallas as pl
from jax.experimental.pallas import tpu as pltpu
from jax.experimental.pallas import tpu_sc as plsc

_B, _S, _D = 16, 512, 2048
_K = 16
_H = 1024
_SF = 128
_SH = 256
_MULT = 1.0

_BBLK = 2
_NB = _B // _BBLK

_NW = 32
_ROWS = _B * _S
_RPW = _ROWS // _NW
_CH = 16
_NCH = _RPW // _CH


def _sc_copy_kernel(emb_hbm, out_hbm, buf, s_in0, s_in1, s_out0, s_out1):
    wid = lax.axis_index("s") * 2 + lax.axis_index("c")
    base = wid * _RPW
    in_sems = (s_in0, s_in1)
    out_sems = (s_out0, s_out1)
    cin = [pltpu.make_async_copy(emb_hbm.at[pl.ds(base + i * _CH, _CH)],
                                 buf.at[i % 2], in_sems[i % 2])
           for i in range(_NCH)]
    cout = [pltpu.make_async_copy(buf.at[i % 2],
                                  out_hbm.at[pl.ds(base + i * _CH, _CH)],
                                  out_sems[i % 2])
            for i in range(_NCH)]
    cin[0].start()
    for i in range(_NCH):
        if i + 1 < _NCH:
            if i >= 1:
                cout[i - 1].wait()
            cin[i + 1].start()
        cin[i].wait()
        cout[i].start()
    cout[_NCH - 2].wait()
    cout[_NCH - 1].wait()


_sc_copy = functools.partial(
    pl.kernel,
    out_type=jax.ShapeDtypeStruct((_ROWS, _D), jnp.float32),
    mesh=plsc.VectorSubcoreMesh(core_axis_name="c", subcore_axis_name="s"),
    scratch_types=[
        pltpu.VMEM((2, _CH, _D), jnp.float32),
        pltpu.SemaphoreType.DMA,
        pltpu.SemaphoreType.DMA,
        pltpu.SemaphoreType.DMA,
        pltpu.SemaphoreType.DMA,
    ],
)(_sc_copy_kernel)


def _pool_kernel(seq_ref, emb_ref, t_ref, W1_ref, b1_ref, W3_ref, b3_ref,
                 slot_ref, b2_ref, b4_ref,
                 h_ref, hs_ref, const_ref, pooled_ref):
    g = pl.program_id(0)
    rows = lax.broadcasted_iota(jnp.int32, (_S, 1), 0)
    for i in range(_BBLK):
        seq = seq_ref[g * _BBLK + i]
        w = (rows < seq).astype(jnp.float32)
        psum = jnp.sum(emb_ref[i] * w, axis=0, keepdims=True)
        inv = 1.0 / jnp.maximum(seq, 1).astype(jnp.float32)
        pooled_ref[pl.ds(g * _BBLK + i, 1), :] = psum * inv

    @pl.when(g == 0)
    def _():
        half = _SF // 2
        t = t_ref[...].reshape(_B, 1)
        io = lax.broadcasted_iota(jnp.int32, (1, half), 1).astype(jnp.float32)
        freqs = jnp.exp((-math.log(10000.0) / half) * io)
        ang = t * freqs
        feat = jnp.concatenate([jnp.cos(ang), jnp.sin(ang)], axis=1)
        pre_s = jnp.dot(feat, W3_ref[...],
                        preferred_element_type=jnp.float32) + b3_ref[...][None, :]
        hs_ref[...] = (pre_s / (1.0 + jnp.exp(-pre_s))).astype(jnp.bfloat16)
        const_ref[...] = (slot_ref[...] + b2_ref[...].reshape(_K, _D)
                          + b4_ref[...].reshape(_K, _D))

    @pl.when(g == _NB - 1)
    def _():
        pre = jnp.dot(pooled_ref[...], W1_ref[...],
                      preferred_element_type=jnp.float32) + b1_ref[...][None, :]
        gelu = 0.5 * pre * (1.0 + lax.erf(pre * (1.0 / math.sqrt(2.0))))
        h_ref[...] = gelu.astype(jnp.bfloat16)


def _mlp_kernel(h_ref, hs_ref, const_ref, W2_ref, W4_ref, post_ref):
    k = pl.program_id(0)
    val = (jnp.dot(h_ref[...], W2_ref[...].astype(jnp.bfloat16),
                   preferred_element_type=jnp.float32)
           + jnp.dot(hs_ref[...], W4_ref[...].astype(jnp.bfloat16),
                     preferred_element_type=jnp.float32)
           + const_ref[pl.ds(k, 1), :]) * _MULT
    post_ref[:, pl.ds(k, 1), :] = val[:, None, :]


def _splice_kernel(outbuf_ref, post_ref, out_ref):
    out_ref[...] = post_ref[...]


def kernel(crossattn_emb, crossattn_seqlens, timesteps,
           W1, b1, W2, b2, slot_embed, W3, b3, W4, b4):
    seq_i32 = crossattn_seqlens.astype(jnp.int32)

    out0 = _sc_copy(crossattn_emb.reshape(_ROWS, _D)).reshape(_B, _S, _D)

    h, hs, const = pl.pallas_call(
        _pool_kernel,
        grid=(_NB,),
        in_specs=[
            pl.BlockSpec(memory_space=pltpu.SMEM),
            pl.BlockSpec((_BBLK, _S, _D), lambda g: (g, 0, 0)),
            pl.BlockSpec((_B,), lambda g: (0,)),
            pl.BlockSpec((_D, _H), lambda g: (0, 0)),
            pl.BlockSpec((_H,), lambda g: (0,)),
            pl.BlockSpec((_SF, _SH), lambda g: (0, 0)),
            pl.BlockSpec((_SH,), lambda g: (0,)),
            pl.BlockSpec((_K, _D), lambda g: (0, 0)),
            pl.BlockSpec((_K * _D,), lambda g: (0,)),
            pl.BlockSpec((_K * _D,), lambda g: (0,)),
        ],
        out_specs=[
            pl.BlockSpec((_B, _H), lambda g: (0, 0)),
            pl.BlockSpec((_B, _SH), lambda g: (0, 0)),
            pl.BlockSpec((_K, _D), lambda g: (0, 0)),
        ],
        out_shape=[
            jax.ShapeDtypeStruct((_B, _H), jnp.bfloat16),
            jax.ShapeDtypeStruct((_B, _SH), jnp.bfloat16),
            jax.ShapeDtypeStruct((_K, _D), jnp.float32),
        ],
        scratch_shapes=[
            pltpu.VMEM((_B, _D), jnp.float32),
        ],
        compiler_params=pltpu.CompilerParams(
            dimension_semantics=("arbitrary",)),
    )(seq_i32, crossattn_emb, timesteps.astype(jnp.float32),
      W1, b1, W3, b3, slot_embed, b2, b4)

    postfix = pl.pallas_call(
        _mlp_kernel,
        grid=(_K,),
        in_specs=[
            pl.BlockSpec((_B, _H), lambda k: (0, 0)),
            pl.BlockSpec((_B, _SH), lambda k: (0, 0)),
            pl.BlockSpec((_K, _D), lambda k: (0, 0)),
            pl.BlockSpec((_H, _D), lambda k: (0, k)),
            pl.BlockSpec((_SH, _D), lambda k: (0, k)),
        ],
        out_specs=pl.BlockSpec((_B, _K, _D), lambda k: (0, 0, 0)),
        out_shape=jax.ShapeDtypeStruct((_B, _K, _D), jnp.float32),
        compiler_params=pltpu.CompilerParams(
            dimension_semantics=("arbitrary",)),
    )(h, hs, const, W2, W4)

    out = pl.pallas_call(
        _splice_kernel,
        grid=(1,),
        in_specs=[
            pl.BlockSpec((_B, _K, _D), lambda i: (0, (_S - _K) // _K, 0)),
            pl.BlockSpec((_B, _K, _D), lambda i: (0, 0, 0)),
        ],
        out_specs=pl.BlockSpec((_B, _K, _D), lambda i: (0, (_S - _K) // _K, 0)),
        out_shape=jax.ShapeDtypeStruct((_B, _S, _D), jnp.float32),
        input_output_aliases={0: 0},
    )(out0, postfix)
    return out


# speedup vs baseline: 1.3956x; 1.3956x over previous
import math

import jax
import jax.numpy as jnp
from jax import lax
from jax.experimental import pallas as pl
from jax.experimental.pallas import tpu as pltpu

_B, _S, _D = 16, 512, 2048
_K = 16
_H = 1024
_SF = 128
_SH = 256
_MULT = 1.0

_BBLK = 1
_NB = _B // _BBLK


def _copy_pool_kernel(seq_ref, emb_ref, t_ref, W1_ref, b1_ref, W3_ref, b3_ref,
                      slot_ref, b2_ref, b4_ref,
                      out_ref, h_ref, hs_ref, const_ref, pooled_ref):
    g = pl.program_id(0)
    out_ref[...] = emb_ref[...]
    rows = lax.broadcasted_iota(jnp.int32, (_S, 1), 0)
    for i in range(_BBLK):
        seq = seq_ref[g * _BBLK + i]
        w = (rows < seq).astype(jnp.float32)
        psum = jnp.sum(emb_ref[i] * w, axis=0, keepdims=True)
        inv = 1.0 / jnp.maximum(seq, 1).astype(jnp.float32)
        pooled_ref[pl.ds(g * _BBLK + i, 1), :] = psum * inv

    @pl.when(g == 0)
    def _():
        half = _SF // 2
        t = t_ref[...].reshape(_B, 1)
        io = lax.broadcasted_iota(jnp.int32, (1, half), 1).astype(jnp.float32)
        freqs = jnp.exp((-math.log(10000.0) / half) * io)
        ang = t * freqs
        feat = jnp.concatenate([jnp.cos(ang), jnp.sin(ang)], axis=1)
        pre_s = jnp.dot(feat, W3_ref[...],
                        preferred_element_type=jnp.float32) + b3_ref[...][None, :]
        hs_ref[...] = (pre_s / (1.0 + jnp.exp(-pre_s))).astype(jnp.bfloat16)
        const_ref[...] = (slot_ref[...] + b2_ref[...].reshape(_K, _D)
                          + b4_ref[...].reshape(_K, _D))

    @pl.when(g == _NB - 1)
    def _():
        pre = jnp.dot(pooled_ref[...], W1_ref[...],
                      preferred_element_type=jnp.float32) + b1_ref[...][None, :]
        gelu = 0.5 * pre * (1.0 + lax.erf(pre * (1.0 / math.sqrt(2.0))))
        h_ref[...] = gelu.astype(jnp.bfloat16)


def _mlp_splice_kernel(outbuf_ref, h_ref, hs_ref, const_ref, W2_ref, W4_ref,
                       out_ref):
    k = pl.program_id(0)
    val = (jnp.dot(h_ref[...], W2_ref[...].astype(jnp.bfloat16),
                   preferred_element_type=jnp.float32)
           + jnp.dot(hs_ref[...], W4_ref[...].astype(jnp.bfloat16),
                     preferred_element_type=jnp.float32)
           + const_ref[pl.ds(k, 1), :]) * _MULT
    out_ref[:, pl.ds(k, 1), :] = val[:, None, :]


def kernel(crossattn_emb, crossattn_seqlens, timesteps,
           W1, b1, W2, b2, slot_embed, W3, b3, W4, b4):
    seq_i32 = crossattn_seqlens.astype(jnp.int32)

    out0, h, hs, const = pl.pallas_call(
        _copy_pool_kernel,
        grid=(_NB,),
        in_specs=[
            pl.BlockSpec(memory_space=pltpu.SMEM),
            pl.BlockSpec((_BBLK, _S, _D), lambda g: (g, 0, 0)),
            pl.BlockSpec((_B,), lambda g: (0,)),
            pl.BlockSpec((_D, _H), lambda g: (0, 0)),
            pl.BlockSpec((_H,), lambda g: (0,)),
            pl.BlockSpec((_SF, _SH), lambda g: (0, 0)),
            pl.BlockSpec((_SH,), lambda g: (0,)),
            pl.BlockSpec((_K, _D), lambda g: (0, 0)),
            pl.BlockSpec((_K * _D,), lambda g: (0,)),
            pl.BlockSpec((_K * _D,), lambda g: (0,)),
        ],
        out_specs=[
            pl.BlockSpec((_BBLK, _S, _D), lambda g: (g, 0, 0)),
            pl.BlockSpec((_B, _H), lambda g: (0, 0)),
            pl.BlockSpec((_B, _SH), lambda g: (0, 0)),
            pl.BlockSpec((_K, _D), lambda g: (0, 0)),
        ],
        out_shape=[
            jax.ShapeDtypeStruct((_B, _S, _D), jnp.float32),
            jax.ShapeDtypeStruct((_B, _H), jnp.bfloat16),
            jax.ShapeDtypeStruct((_B, _SH), jnp.bfloat16),
            jax.ShapeDtypeStruct((_K, _D), jnp.float32),
        ],
        scratch_shapes=[
            pltpu.VMEM((_B, _D), jnp.float32),
        ],
        compiler_params=pltpu.CompilerParams(
            dimension_semantics=("arbitrary",)),
    )(seq_i32, crossattn_emb, timesteps.astype(jnp.float32),
      W1, b1, W3, b3, slot_embed, b2, b4)

    out = pl.pallas_call(
        _mlp_splice_kernel,
        grid=(_K,),
        in_specs=[
            pl.BlockSpec((_B, _K, _D), lambda k: (0, (_S - _K) // _K, 0)),
            pl.BlockSpec((_B, _H), lambda k: (0, 0)),
            pl.BlockSpec((_B, _SH), lambda k: (0, 0)),
            pl.BlockSpec((_K, _D), lambda k: (0, 0)),
            pl.BlockSpec((_H, _D), lambda k: (0, k)),
            pl.BlockSpec((_SH, _D), lambda k: (0, k)),
        ],
        out_specs=pl.BlockSpec((_B, _K, _D), lambda k: (0, (_S - _K) // _K, 0)),
        out_shape=jax.ShapeDtypeStruct((_B, _S, _D), jnp.float32),
        input_output_aliases={0: 0},
        compiler_params=pltpu.CompilerParams(
            dimension_semantics=("arbitrary",)),
    )(out0, h, hs, const, W2, W4)
    return out


# speedup vs baseline: 1.4091x; 1.0097x over previous
import math

import jax
import jax.numpy as jnp
from jax import lax
from jax.experimental import pallas as pl
from jax.experimental.pallas import tpu as pltpu

_B, _S, _D = 16, 512, 2048
_K = 16
_H = 1024
_SF = 128
_SH = 256
_MULT = 1.0

_BBLK = 2
_NB = _B // _BBLK


def _copy_pool_kernel(seq_ref, emb_ref, t_ref, W1_ref, b1_ref, W3_ref, b3_ref,
                      slot_ref, b2_ref, b4_ref,
                      out_ref, h_ref, hs_ref, const_ref, pooled_ref):
    g = pl.program_id(0)
    out_ref[...] = emb_ref[...]
    rows = lax.broadcasted_iota(jnp.int32, (_S, 1), 0)
    for i in range(_BBLK):
        seq = seq_ref[g * _BBLK + i]
        w = (rows < seq).astype(jnp.float32)
        psum = jnp.sum(emb_ref[i] * w, axis=0, keepdims=True)
        inv = 1.0 / jnp.maximum(seq, 1).astype(jnp.float32)
        pooled_ref[pl.ds(g * _BBLK + i, 1), :] = psum * inv

    @pl.when(g == 0)
    def _():
        half = _SF // 2
        t = t_ref[...].reshape(_B, 1)
        io = lax.broadcasted_iota(jnp.int32, (1, half), 1).astype(jnp.float32)
        freqs = jnp.exp((-math.log(10000.0) / half) * io)
        ang = t * freqs
        feat = jnp.concatenate([jnp.cos(ang), jnp.sin(ang)], axis=1)
        pre_s = jnp.dot(feat, W3_ref[...],
                        preferred_element_type=jnp.float32) + b3_ref[...][None, :]
        hs_ref[...] = (pre_s / (1.0 + jnp.exp(-pre_s))).astype(jnp.bfloat16)
        const_ref[...] = (slot_ref[...] + b2_ref[...].reshape(_K, _D)
                          + b4_ref[...].reshape(_K, _D))

    @pl.when(g == _NB - 1)
    def _():
        pre = jnp.dot(pooled_ref[...], W1_ref[...],
                      preferred_element_type=jnp.float32) + b1_ref[...][None, :]
        gelu = 0.5 * pre * (1.0 + lax.erf(pre * (1.0 / math.sqrt(2.0))))
        h_ref[...] = gelu.astype(jnp.bfloat16)


def _mlp_splice_kernel(outbuf_ref, h_ref, hs_ref, const_ref, W2_ref, W4_ref,
                       out_ref):
    k = pl.program_id(0)
    val = (jnp.dot(h_ref[...], W2_ref[...].astype(jnp.bfloat16),
                   preferred_element_type=jnp.float32)
           + jnp.dot(hs_ref[...], W4_ref[...].astype(jnp.bfloat16),
                     preferred_element_type=jnp.float32)
           + const_ref[pl.ds(k, 1), :]) * _MULT
    out_ref[:, pl.ds(k, 1), :] = val[:, None, :]


def kernel(crossattn_emb, crossattn_seqlens, timesteps,
           W1, b1, W2, b2, slot_embed, W3, b3, W4, b4):
    seq_i32 = crossattn_seqlens.astype(jnp.int32)

    out0, h, hs, const = pl.pallas_call(
        _copy_pool_kernel,
        grid=(_NB,),
        in_specs=[
            pl.BlockSpec(memory_space=pltpu.SMEM),
            pl.BlockSpec((_BBLK, _S, _D), lambda g: (g, 0, 0)),
            pl.BlockSpec((_B,), lambda g: (0,)),
            pl.BlockSpec((_D, _H), lambda g: (0, 0)),
            pl.BlockSpec((_H,), lambda g: (0,)),
            pl.BlockSpec((_SF, _SH), lambda g: (0, 0)),
            pl.BlockSpec((_SH,), lambda g: (0,)),
            pl.BlockSpec((_K, _D), lambda g: (0, 0)),
            pl.BlockSpec((_K * _D,), lambda g: (0,)),
            pl.BlockSpec((_K * _D,), lambda g: (0,)),
        ],
        out_specs=[
            pl.BlockSpec((_BBLK, _S, _D), lambda g: (g, 0, 0)),
            pl.BlockSpec((_B, _H), lambda g: (0, 0)),
            pl.BlockSpec((_B, _SH), lambda g: (0, 0)),
            pl.BlockSpec((_K, _D), lambda g: (0, 0)),
        ],
        out_shape=[
            jax.ShapeDtypeStruct((_B, _S, _D), jnp.float32),
            jax.ShapeDtypeStruct((_B, _H), jnp.bfloat16),
            jax.ShapeDtypeStruct((_B, _SH), jnp.bfloat16),
            jax.ShapeDtypeStruct((_K, _D), jnp.float32),
        ],
        scratch_shapes=[
            pltpu.VMEM((_B, _D), jnp.float32),
        ],
        compiler_params=pltpu.CompilerParams(
            dimension_semantics=("arbitrary",)),
    )(seq_i32, crossattn_emb, timesteps.astype(jnp.float32),
      W1, b1, W3, b3, slot_embed, b2, b4)

    out = pl.pallas_call(
        _mlp_splice_kernel,
        grid=(_K,),
        in_specs=[
            pl.BlockSpec((_B, _K, _D), lambda k: (0, (_S - _K) // _K, 0)),
            pl.BlockSpec((_B, _H), lambda k: (0, 0)),
            pl.BlockSpec((_B, _SH), lambda k: (0, 0)),
            pl.BlockSpec((_K, _D), lambda k: (0, 0)),
            pl.BlockSpec((_H, _D), lambda k: (0, k)),
            pl.BlockSpec((_SH, _D), lambda k: (0, k)),
        ],
        out_specs=pl.BlockSpec((_B, _K, _D), lambda k: (0, (_S - _K) // _K, 0)),
        out_shape=jax.ShapeDtypeStruct((_B, _S, _D), jnp.float32),
        input_output_aliases={0: 0},
        compiler_params=pltpu.CompilerParams(
            dimension_semantics=("arbitrary",)),
    )(out0, h, hs, const, W2, W4)
    return out
